# baseline (speedup 1.0000x reference)
import functools

import jax
import jax.numpy as jnp
from jax import lax
from jax.experimental import pallas as pl
from jax.experimental.pallas import tpu as pltpu
from jax.experimental.pallas import tpu_sc as plsc

BATCH = 16384
EMBED_DIM = 32
N_USERS = 1000000
N_MOVIES = 100000
NUM_CORES = 2
NUM_SUBCORES = 16
NUM_WORKERS = NUM_CORES * NUM_SUBCORES
BPW = BATCH // NUM_WORKERS
CHUNK = 128
NCHUNK = BPW // CHUNK
LANES = 16


_TC_BLK = 131072


def _colsum_body(t_ref, w_ref, o_ref):
    o_ref[...] = jnp.sum(t_ref[...] * w_ref[...], axis=0)


def _weighted_colsum(table_t, w_col, n):
    grid = (n + _TC_BLK - 1) // _TC_BLK
    return pl.pallas_call(
        _colsum_body,
        grid=(grid,),
        in_specs=[
            pl.BlockSpec((EMBED_DIM, _TC_BLK), lambda i: (0, i)),
            pl.BlockSpec((EMBED_DIM, 1), lambda i: (0, 0)),
        ],
        out_specs=pl.BlockSpec((_TC_BLK,), lambda i: (i,)),
        out_shape=jax.ShapeDtypeStruct((n,), jnp.float32),
    )(table_t, w_col)


_mesh = plsc.VectorSubcoreMesh(
    core_axis_name="c", subcore_axis_name="s", num_cores=NUM_CORES,
    num_subcores=NUM_SUBCORES)


@functools.partial(
    pl.kernel,
    out_type=jax.ShapeDtypeStruct((BATCH,), jnp.float32),
    mesh=_mesh,
    compiler_params=pltpu.CompilerParams(needs_layout_passes=False,
                                         use_tc_tiling_on_sc=False),
    scratch_types=[
        pltpu.VMEM((BPW,), jnp.int32),
        pltpu.VMEM((BPW,), jnp.int32),
        pltpu.VMEM((BPW,), jnp.float32),
        pltpu.VMEM((BPW,), jnp.float32),
        pltpu.VMEM((LANES,), jnp.float32),
        pltpu.VMEM((BPW,), jnp.float32),
        pltpu.SemaphoreType.DMA,
        pltpu.SemaphoreType.DMA,
    ],
)
def _gather_add(users_hbm, movies_hbm, uw_hbm, mw_hbm, b_hbm, out_hbm,
                uidx, midx, gu, gm, bvec, outv, sem_u, sem_m):
    wid = lax.axis_index("s") * NUM_CORES + lax.axis_index("c")
    base = wid * BPW
    pltpu.sync_copy(users_hbm.at[pl.ds(base, BPW)], uidx)
    pltpu.sync_copy(movies_hbm.at[pl.ds(base, BPW)], midx)
    pltpu.sync_copy(b_hbm, bvec)
    copies = []
    for c in range(NCHUNK):
        sl = pl.ds(c * CHUNK, CHUNK)
        copies.append(pltpu.async_copy(uw_hbm.at[uidx.at[sl]], gu.at[sl],
                                       sem_u))
        copies.append(pltpu.async_copy(mw_hbm.at[midx.at[sl]], gm.at[sl],
                                       sem_m))
    for cp in copies:
        cp.wait()
    b_val = bvec[...]
    for s in range(BPW // LANES):
        sl = pl.ds(s * LANES, LANES)
        outv[sl] = gu[sl] + gm[sl] + b_val
    pltpu.sync_copy(outv, out_hbm.at[pl.ds(base, BPW)])


def kernel(users, movies, user_table, movie_table, W, b):
    w = W.reshape(-1)
    uw = _weighted_colsum(user_table.T, w[:EMBED_DIM].reshape(EMBED_DIM, 1),
                          N_USERS)
    mw = _weighted_colsum(movie_table.T, w[EMBED_DIM:].reshape(EMBED_DIM, 1),
                          N_MOVIES)
    bvec = jnp.broadcast_to(b.reshape(()), (LANES,))
    out = _gather_add(users.astype(jnp.int32), movies.astype(jnp.int32),
                      uw, mw, bvec)
    return out.reshape(BATCH, 1)


# speedup vs baseline: 1.0197x; 1.0197x over previous
import functools

import jax
import jax.numpy as jnp
from jax import lax
from jax.experimental import pallas as pl
from jax.experimental.pallas import tpu as pltpu
from jax.experimental.pallas import tpu_sc as plsc

BATCH = 16384
EMBED_DIM = 32
N_USERS = 1000000
N_MOVIES = 100000
NUM_CORES = 2
NUM_SUBCORES = 16
NUM_WORKERS = NUM_CORES * NUM_SUBCORES
BPW = BATCH // NUM_WORKERS
CHUNK = 128
NCHUNK = BPW // CHUNK
LANES = 16


_TC_BLK = 65536


def _colsum_body(t_ref, w_ref, o_ref):
    o_ref[...] = jnp.sum(t_ref[...] * w_ref[...], axis=0)


def _weighted_colsum(table_t, w_col, n):
    grid = (n + _TC_BLK - 1) // _TC_BLK
    return pl.pallas_call(
        _colsum_body,
        grid=(grid,),
        in_specs=[
            pl.BlockSpec((EMBED_DIM, _TC_BLK), lambda i: (0, i)),
            pl.BlockSpec((EMBED_DIM, 1), lambda i: (0, 0)),
        ],
        out_specs=pl.BlockSpec((_TC_BLK,), lambda i: (i,)),
        out_shape=jax.ShapeDtypeStruct((n,), jnp.float32),
    )(table_t, w_col)


_mesh = plsc.VectorSubcoreMesh(
    core_axis_name="c", subcore_axis_name="s", num_cores=NUM_CORES,
    num_subcores=NUM_SUBCORES)


@functools.partial(
    pl.kernel,
    out_type=jax.ShapeDtypeStruct((BATCH,), jnp.float32),
    mesh=_mesh,
    compiler_params=pltpu.CompilerParams(needs_layout_passes=False,
                                         use_tc_tiling_on_sc=False),
    scratch_types=[
        pltpu.VMEM((BPW,), jnp.int32),
        pltpu.VMEM((BPW,), jnp.int32),
        pltpu.VMEM((BPW,), jnp.float32),
        pltpu.VMEM((BPW,), jnp.float32),
        pltpu.VMEM((LANES,), jnp.float32),
        pltpu.VMEM((BPW,), jnp.float32),
        pltpu.SemaphoreType.DMA,
        pltpu.SemaphoreType.DMA,
    ],
)
def _gather_add(users_hbm, movies_hbm, uw_hbm, mw_hbm, b_hbm, out_hbm,
                uidx, midx, gu, gm, bvec, outv, sem_u, sem_m):
    wid = lax.axis_index("s") * NUM_CORES + lax.axis_index("c")
    base = wid * BPW
    pltpu.sync_copy(users_hbm.at[pl.ds(base, BPW)], uidx)
    pltpu.sync_copy(movies_hbm.at[pl.ds(base, BPW)], midx)
    pltpu.sync_copy(b_hbm, bvec)
    copies = []
    for c in range(NCHUNK):
        sl = pl.ds(c * CHUNK, CHUNK)
        copies.append(pltpu.async_copy(uw_hbm.at[uidx.at[sl]], gu.at[sl],
                                       sem_u))
        copies.append(pltpu.async_copy(mw_hbm.at[midx.at[sl]], gm.at[sl],
                                       sem_m))
    for cp in copies:
        cp.wait()
    b_val = bvec[...]
    for s in range(BPW // LANES):
        sl = pl.ds(s * LANES, LANES)
        outv[sl] = gu[sl] + gm[sl] + b_val
    pltpu.sync_copy(outv, out_hbm.at[pl.ds(base, BPW)])


def kernel(users, movies, user_table, movie_table, W, b):
    w = W.reshape(-1)
    uw = _weighted_colsum(user_table.T, w[:EMBED_DIM].reshape(EMBED_DIM, 1),
                          N_USERS)
    mw = _weighted_colsum(movie_table.T, w[EMBED_DIM:].reshape(EMBED_DIM, 1),
                          N_MOVIES)
    bvec = jnp.broadcast_to(b.reshape(()), (LANES,))
    out = _gather_add(users.astype(jnp.int32), movies.astype(jnp.int32),
                      uw, mw, bvec)
    return out.reshape(BATCH, 1)


# speedup vs baseline: 1.0651x; 1.0446x over previous
import functools

import jax
import jax.numpy as jnp
from jax import lax
from jax.experimental import pallas as pl
from jax.experimental.pallas import tpu as pltpu
from jax.experimental.pallas import tpu_sc as plsc

BATCH = 16384
EMBED_DIM = 32
N_USERS = 1000000
N_MOVIES = 100000
NUM_CORES = 2
NUM_SUBCORES = 16
NUM_WORKERS = NUM_CORES * NUM_SUBCORES
BPW = BATCH // NUM_WORKERS
CHUNK = 128
NCHUNK = BPW // CHUNK
LANES = 16


_TC_BLK = 8192
_TC_NBUF = 8


def _colsum_body(nfull, rem, t_ref, w_ref, o_ref, bufs, rbuf, sems, rsem):
    def issue(i):
        slot = i % _TC_NBUF
        return pltpu.make_async_copy(
            t_ref.at[:, pl.ds(i * _TC_BLK, _TC_BLK)], bufs.at[slot],
            sems.at[slot])

    for i in range(min(_TC_NBUF, nfull)):
        issue(i).start()
    if rem:
        rcopy = pltpu.make_async_copy(
            t_ref.at[:, pl.ds(nfull * _TC_BLK, rem)], rbuf, rsem)
        rcopy.start()
    w_val = w_ref[...]
    for i in range(nfull):
        slot = i % _TC_NBUF
        issue(i).wait()
        o_ref[pl.ds(i * _TC_BLK, _TC_BLK)] = jnp.sum(
            bufs[slot] * w_val, axis=0)
        if i + _TC_NBUF < nfull:
            issue(i + _TC_NBUF).start()
    if rem:
        rcopy.wait()
        o_ref[pl.ds(nfull * _TC_BLK, rem)] = jnp.sum(rbuf[...] * w_val,
                                                     axis=0)


def _weighted_colsum(table_t, w_col, n):
    nfull, rem = divmod(n, _TC_BLK)
    return pl.pallas_call(
        functools.partial(_colsum_body, nfull, rem),
        in_specs=[
            pl.BlockSpec(memory_space=pltpu.MemorySpace.HBM),
            pl.BlockSpec((EMBED_DIM, 1), lambda: (0, 0)),
        ],
        out_specs=pl.BlockSpec((n,), lambda: (0,)),
        out_shape=jax.ShapeDtypeStruct((n,), jnp.float32),
        scratch_shapes=[
            pltpu.VMEM((_TC_NBUF, EMBED_DIM, _TC_BLK), jnp.float32),
            pltpu.VMEM((EMBED_DIM, max(rem, 1)), jnp.float32),
            pltpu.SemaphoreType.DMA((_TC_NBUF,)),
            pltpu.SemaphoreType.DMA,
        ],
    )(table_t, w_col)


_mesh = plsc.VectorSubcoreMesh(
    core_axis_name="c", subcore_axis_name="s", num_cores=NUM_CORES,
    num_subcores=NUM_SUBCORES)


@functools.partial(
    pl.kernel,
    out_type=jax.ShapeDtypeStruct((BATCH,), jnp.float32),
    mesh=_mesh,
    compiler_params=pltpu.CompilerParams(needs_layout_passes=False,
                                         use_tc_tiling_on_sc=False),
    scratch_types=[
        pltpu.VMEM((BPW,), jnp.int32),
        pltpu.VMEM((BPW,), jnp.int32),
        pltpu.VMEM((BPW,), jnp.float32),
        pltpu.VMEM((BPW,), jnp.float32),
        pltpu.VMEM((LANES,), jnp.float32),
        pltpu.VMEM((BPW,), jnp.float32),
        pltpu.SemaphoreType.DMA,
        pltpu.SemaphoreType.DMA,
    ],
)
def _gather_add(users_hbm, movies_hbm, uw_hbm, mw_hbm, b_hbm, out_hbm,
                uidx, midx, gu, gm, bvec, outv, sem_u, sem_m):
    wid = lax.axis_index("s") * NUM_CORES + lax.axis_index("c")
    base = wid * BPW
    pltpu.sync_copy(users_hbm.at[pl.ds(base, BPW)], uidx)
    pltpu.sync_copy(movies_hbm.at[pl.ds(base, BPW)], midx)
    pltpu.sync_copy(b_hbm, bvec)
    copies = []
    for c in range(NCHUNK):
        sl = pl.ds(c * CHUNK, CHUNK)
        copies.append(pltpu.async_copy(uw_hbm.at[uidx.at[sl]], gu.at[sl],
                                       sem_u))
        copies.append(pltpu.async_copy(mw_hbm.at[midx.at[sl]], gm.at[sl],
                                       sem_m))
    for cp in copies:
        cp.wait()
    b_val = bvec[...]
    for s in range(BPW // LANES):
        sl = pl.ds(s * LANES, LANES)
        outv[sl] = gu[sl] + gm[sl] + b_val
    pltpu.sync_copy(outv, out_hbm.at[pl.ds(base, BPW)])


def kernel(users, movies, user_table, movie_table, W, b):
    w = W.reshape(-1)
    uw = _weighted_colsum(user_table.T, w[:EMBED_DIM].reshape(EMBED_DIM, 1),
                          N_USERS)
    mw = _weighted_colsum(movie_table.T, w[EMBED_DIM:].reshape(EMBED_DIM, 1),
                          N_MOVIES)
    bvec = jnp.broadcast_to(b.reshape(()), (LANES,))
    out = _gather_add(users.astype(jnp.int32), movies.astype(jnp.int32),
                      uw, mw, bvec)
    return out.reshape(BATCH, 1)


# speedup vs baseline: 1.1101x; 1.0422x over previous
import functools

import jax
import jax.numpy as jnp
from jax import lax
from jax.experimental import pallas as pl
from jax.experimental.pallas import tpu as pltpu
from jax.experimental.pallas import tpu_sc as plsc

BATCH = 16384
EMBED_DIM = 32
N_USERS = 1000000
N_MOVIES = 100000
NUM_CORES = 2
NUM_SUBCORES = 16
NUM_WORKERS = NUM_CORES * NUM_SUBCORES
BPW = BATCH // NUM_WORKERS
CHUNK = 128
NCHUNK = BPW // CHUNK
LANES = 16


_TC_BLK = 8192
_TC_NBUF = 8


def _fused_colsum_body(nu, ru, nm, rm, u_ref, m_ref, w_ref,
                       ou_ref, om_ref, bufs, rub, rmb, sems, rus, rms):
    chunks = ([(u_ref, ou_ref, 0, i) for i in range(nu)]
              + [(m_ref, om_ref, 1, i) for i in range(nm)])
    n = len(chunks)

    def issue(k):
        t_ref, _, _, i = chunks[k]
        slot = k % _TC_NBUF
        return pltpu.make_async_copy(
            t_ref.at[:, pl.ds(i * _TC_BLK, _TC_BLK)], bufs.at[slot],
            sems.at[slot])

    for k in range(min(_TC_NBUF, n)):
        issue(k).start()
    rucopy = pltpu.make_async_copy(
        u_ref.at[:, pl.ds(nu * _TC_BLK, ru)], rub, rus)
    rucopy.start()
    rmcopy = pltpu.make_async_copy(
        m_ref.at[:, pl.ds(nm * _TC_BLK, rm)], rmb, rms)
    rmcopy.start()
    wvals = [w_ref[:, 0:1], w_ref[:, 1:2]]
    for k in range(n):
        _, o_ref, wsel, i = chunks[k]
        issue(k).wait()
        o_ref[pl.ds(i * _TC_BLK, _TC_BLK)] = jnp.sum(
            bufs[k % _TC_NBUF] * wvals[wsel], axis=0)
        if k + _TC_NBUF < n:
            issue(k + _TC_NBUF).start()
    rucopy.wait()
    ou_ref[pl.ds(nu * _TC_BLK, ru)] = jnp.sum(rub[...] * wvals[0], axis=0)
    rmcopy.wait()
    om_ref[pl.ds(nm * _TC_BLK, rm)] = jnp.sum(rmb[...] * wvals[1], axis=0)


def _fused_colsum(user_t, movie_t, w_mat):
    nu, ru = divmod(N_USERS, _TC_BLK)
    nm, rm = divmod(N_MOVIES, _TC_BLK)
    return pl.pallas_call(
        functools.partial(_fused_colsum_body, nu, ru, nm, rm),
        in_specs=[
            pl.BlockSpec(memory_space=pltpu.MemorySpace.HBM),
            pl.BlockSpec(memory_space=pltpu.MemorySpace.HBM),
            pl.BlockSpec((EMBED_DIM, 2), lambda: (0, 0)),
        ],
        out_specs=[
            pl.BlockSpec((N_USERS,), lambda: (0,)),
            pl.BlockSpec((N_MOVIES,), lambda: (0,)),
        ],
        out_shape=[
            jax.ShapeDtypeStruct((N_USERS,), jnp.float32),
            jax.ShapeDtypeStruct((N_MOVIES,), jnp.float32),
        ],
        scratch_shapes=[
            pltpu.VMEM((_TC_NBUF, EMBED_DIM, _TC_BLK), jnp.float32),
            pltpu.VMEM((EMBED_DIM, ru), jnp.float32),
            pltpu.VMEM((EMBED_DIM, rm), jnp.float32),
            pltpu.SemaphoreType.DMA((_TC_NBUF,)),
            pltpu.SemaphoreType.DMA,
            pltpu.SemaphoreType.DMA,
        ],
    )(user_t, movie_t, w_mat)


_mesh = plsc.VectorSubcoreMesh(
    core_axis_name="c", subcore_axis_name="s", num_cores=NUM_CORES,
    num_subcores=NUM_SUBCORES)


@functools.partial(
    pl.kernel,
    out_type=jax.ShapeDtypeStruct((BATCH,), jnp.float32),
    mesh=_mesh,
    compiler_params=pltpu.CompilerParams(needs_layout_passes=False,
                                         use_tc_tiling_on_sc=False),
    scratch_types=[
        pltpu.VMEM((BPW,), jnp.int32),
        pltpu.VMEM((BPW,), jnp.int32),
        pltpu.VMEM((BPW,), jnp.float32),
        pltpu.VMEM((BPW,), jnp.float32),
        pltpu.VMEM((LANES,), jnp.float32),
        pltpu.VMEM((BPW,), jnp.float32),
        pltpu.SemaphoreType.DMA,
        pltpu.SemaphoreType.DMA,
    ],
)
def _gather_add(users_hbm, movies_hbm, uw_hbm, mw_hbm, b_hbm, out_hbm,
                uidx, midx, gu, gm, bvec, outv, sem_u, sem_m):
    wid = lax.axis_index("s") * NUM_CORES + lax.axis_index("c")
    base = wid * BPW
    pltpu.sync_copy(users_hbm.at[pl.ds(base, BPW)], uidx)
    pltpu.sync_copy(movies_hbm.at[pl.ds(base, BPW)], midx)
    pltpu.sync_copy(b_hbm, bvec)
    copies = []
    for c in range(NCHUNK):
        sl = pl.ds(c * CHUNK, CHUNK)
        copies.append(pltpu.async_copy(uw_hbm.at[uidx.at[sl]], gu.at[sl],
                                       sem_u))
        copies.append(pltpu.async_copy(mw_hbm.at[midx.at[sl]], gm.at[sl],
                                       sem_m))
    for cp in copies:
        cp.wait()
    b_val = bvec[...]
    for s in range(BPW // LANES):
        sl = pl.ds(s * LANES, LANES)
        outv[sl] = gu[sl] + gm[sl] + b_val
    pltpu.sync_copy(outv, out_hbm.at[pl.ds(base, BPW)])


def kernel(users, movies, user_table, movie_table, W, b):
    w = W.reshape(-1)
    w_mat = jnp.stack([w[:EMBED_DIM], w[EMBED_DIM:]], axis=1)
    uw, mw = _fused_colsum(user_table.T, movie_table.T, w_mat)
    bvec = jnp.broadcast_to(b.reshape(()), (LANES,))
    out = _gather_add(users.astype(jnp.int32), movies.astype(jnp.int32),
                      uw, mw, bvec)
    return out.reshape(BATCH, 1)
